# baseline (speedup 1.0000x reference)
import functools

import jax
import jax.numpy as jnp
from jax import lax
from jax.experimental import pallas as pl
from jax.experimental.pallas import tpu as pltpu
from jax.experimental.pallas import tpu_sc as plsc

NS = 16
NSC = 2
RB = 16
CH0 = 128
CH1 = 32


def _tc_body(x_ref, w_ref, s_ref, b_ref, y_ref, z_ref):
    xb = x_ref[...]
    y_ref[...] = jnp.dot(xb, w_ref[...], preferred_element_type=jnp.float32)
    z_ref[...] = jnp.dot(xb, s_ref[...], preferred_element_type=jnp.float32) + b_ref[...]


def _tc_prep(x, weight, self_loops, bias):
    N, D = x.shape
    R = 1000
    grid = (N // R,)
    return pl.pallas_call(
        _tc_body,
        grid=grid,
        in_specs=[
            pl.BlockSpec((R, D), lambda i: (i, 0)),
            pl.BlockSpec((D, D), lambda i: (0, 0)),
            pl.BlockSpec((D, D), lambda i: (0, 0)),
            pl.BlockSpec((1, D), lambda i: (0, 0)),
        ],
        out_specs=[
            pl.BlockSpec((R, D), lambda i: (i, 0)),
            pl.BlockSpec((R, D), lambda i: (i, 0)),
        ],
        out_shape=[
            jax.ShapeDtypeStruct((N, D), jnp.float32),
            jax.ShapeDtypeStruct((N, D), jnp.float32),
        ],
    )(x, weight, self_loops, bias.reshape(1, D))


def _make_sc_kernel(N, D):
    rows = -(-N // (NS * 8)) * 8
    NACC = rows * NS
    full, rem = divmod(rows, 128)
    lastz = N - (NS - 1) * rows
    mesh = plsc.VectorSubcoreMesh(core_axis_name="c", subcore_axis_name="s")

    @functools.partial(
        pl.kernel,
        out_type=jax.ShapeDtypeStruct((NSC, NACC, D), jnp.float32),
        mesh=mesh,
        scratch_types=[
            pltpu.VMEM((2, RB, 128), jnp.int32),
            pltpu.VMEM((2, RB, 128), jnp.int32),
            pltpu.VMEM((2, 128, D), jnp.float32),
            pltpu.VMEM_SHARED((NACC, D), jnp.float32),
            pltpu.SemaphoreType.DMA,
            pltpu.SemaphoreType.DMA,
            pltpu.SemaphoreType.DMA,
        ],
    )
    def sc_fn(y_hbm, z_hbm, zero_hbm, src_hbm, dst_hbm, out_hbm,
              src_v, dst_v, gbuf, acc_sh, gsem, ssem, rsem):
        c = lax.axis_index("c")
        s = lax.axis_index("s")
        coff = jnp.where(c == 0, s * CH0, NS * CH0 + s * CH1)
        chg = jnp.where(c == 0, CH0 // RB, CH1 // RB)

        def drain(dst, sem):
            pltpu.make_async_copy(y_hbm.at[pl.ds(0, dst.shape[0])], dst, sem).wait()

        @pl.when(c == 0)
        def _():
            @pl.when(s < NS - 1)
            def _():
                pltpu.sync_copy(z_hbm.at[pl.ds(s * rows, rows)],
                                acc_sh.at[pl.ds(s * rows, rows)])

            @pl.when(s == NS - 1)
            def _():
                pltpu.sync_copy(z_hbm.at[pl.ds((NS - 1) * rows, lastz)],
                                acc_sh.at[pl.ds((NS - 1) * rows, lastz)])

        @pl.when(c == 1)
        def _():
            pltpu.sync_copy(zero_hbm, gbuf.at[0])
            for t in range(full):
                pltpu.sync_copy(gbuf.at[0],
                                acc_sh.at[pl.ds(s * rows + t * 128, 128)])
            if rem:
                pltpu.sync_copy(gbuf.at[0, pl.ds(0, rem)],
                                acc_sh.at[pl.ds(s * rows + full * 128, rem)])

        pltpu.async_copy(src_hbm.at[pl.ds(coff, RB)], src_v.at[0], rsem)
        pltpu.async_copy(dst_hbm.at[pl.ds(coff, RB)], dst_v.at[0], rsem)
        plsc.subcore_barrier()

        def group(g, carry):
            p = lax.rem(g, 2)
            pltpu.make_async_copy(src_hbm.at[pl.ds(0, RB)], src_v.at[0], rsem).wait()
            pltpu.make_async_copy(dst_hbm.at[pl.ds(0, RB)], dst_v.at[0], rsem).wait()

            @pl.when(g + 1 < chg)
            def _():
                pn = lax.rem(g + 1, 2)
                pltpu.async_copy(src_hbm.at[pl.ds(coff + (g + 1) * RB, RB)],
                                 src_v.at[pn], rsem)
                pltpu.async_copy(dst_hbm.at[pl.ds(coff + (g + 1) * RB, RB)],
                                 dst_v.at[pn], rsem)

            pltpu.async_copy(y_hbm.at[dst_v.at[p, 0]], gbuf.at[0], gsem)

            def body(j, carry):
                b = lax.rem(j, 2)
                bn = lax.rem(j + 1, 2)

                @pl.when(j > 0)
                def _():
                    drain(gbuf.at[0], ssem)

                @pl.when(j + 1 < RB)
                def _():
                    pltpu.async_copy(y_hbm.at[dst_v.at[p, j + 1]], gbuf.at[bn], gsem)

                drain(gbuf.at[0], gsem)
                pltpu.async_copy(gbuf.at[b], acc_sh.at[src_v.at[p, j]], ssem, add=True)
                return carry

            carry = lax.fori_loop(0, RB, body, carry)
            drain(gbuf.at[0], ssem)
            return carry

        lax.fori_loop(0, chg, group, 0)
        plsc.subcore_barrier()

        pltpu.sync_copy(
            acc_sh.at[pl.ds(s * rows, rows)],
            out_hbm.at[c, pl.ds(s * rows, rows)],
        )

    return sc_fn


def kernel(x, edge_index, weight, self_loops, bias):
    N, D = x.shape
    E = edge_index.shape[0]
    CT = NS * (CH0 + CH1)
    EP = CT * 128
    assert EP >= E
    pad = EP - E

    y, z = _tc_prep(x, weight, self_loops, bias)

    src = edge_index[:, 0]
    dst = edge_index[:, 1]
    srcp = jnp.concatenate([src, jnp.full((pad,), N, jnp.int32)]).reshape(CT, 128)
    dstp = jnp.concatenate([dst, jnp.zeros((pad,), jnp.int32)]).reshape(CT, 128)

    zero_blk = jnp.zeros((128, D), jnp.float32)
    out_sc = _make_sc_kernel(N, D)(y, z, zero_blk, srcp, dstp)
    return out_sc[0, :N] + out_sc[1, :N]


# speedup vs baseline: 1.0116x; 1.0116x over previous
import functools

import jax
import jax.numpy as jnp
from jax import lax
from jax.experimental import pallas as pl
from jax.experimental.pallas import tpu as pltpu
from jax.experimental.pallas import tpu_sc as plsc

NS = 16
NSC = 2
CK = 32
GC = 4


def _tc_body(x_ref, w_ref, s_ref, b_ref, y_ref, z_ref):
    xb = x_ref[...]
    y_ref[...] = jnp.dot(xb, w_ref[...], preferred_element_type=jnp.float32)
    z_ref[...] = jnp.dot(xb, s_ref[...], preferred_element_type=jnp.float32) + b_ref[...]


def _tc_prep(x, weight, self_loops, bias):
    N, D = x.shape
    R = 1000
    grid = (N // R,)
    return pl.pallas_call(
        _tc_body,
        grid=grid,
        in_specs=[
            pl.BlockSpec((R, D), lambda i: (i, 0)),
            pl.BlockSpec((D, D), lambda i: (0, 0)),
            pl.BlockSpec((D, D), lambda i: (0, 0)),
            pl.BlockSpec((1, D), lambda i: (0, 0)),
        ],
        out_specs=[
            pl.BlockSpec((R, D), lambda i: (i, 0)),
            pl.BlockSpec((R, D), lambda i: (i, 0)),
        ],
        out_shape=[
            jax.ShapeDtypeStruct((N, D), jnp.float32),
            jax.ShapeDtypeStruct((N, D), jnp.float32),
        ],
    )(x, weight, self_loops, bias.reshape(1, D))


def _make_sc_kernel(N, D, GROUPS):
    HN = N // 2
    rows = -(-N // (NS * 8)) * 8
    last = N - (NS - 1) * rows
    yrows = -(-HN // (NS * 8)) * 8
    ylast = HN - (NS - 1) * yrows
    mesh = plsc.VectorSubcoreMesh(core_axis_name="c", subcore_axis_name="s")

    @functools.partial(
        pl.kernel,
        out_type=jax.ShapeDtypeStruct((NSC, N, D), jnp.float32),
        mesh=mesh,
        scratch_types=[
            pltpu.VMEM((2, 1, GC, CK), jnp.int32),
            pltpu.VMEM((2, 1, GC, CK), jnp.int32),
            pltpu.VMEM((2, CK, D), jnp.float32),
            pltpu.VMEM_SHARED((HN + 8, D), jnp.float32),
            pltpu.VMEM_SHARED((N, D), jnp.float32),
            pltpu.SemaphoreType.DMA,
            pltpu.SemaphoreType.DMA,
            pltpu.SemaphoreType.DMA,
        ],
    )
    def sc_fn(y_hbm, z_hbm, zero_hbm, src_hbm, dst_hbm, out_hbm,
              src_v, dst_v, gbuf, y_sh, acc_sh, gsem, ssem, rsem):
        c = lax.axis_index("c")
        s = lax.axis_index("s")
        base = s * GROUPS

        def drain_buf(sem):
            pltpu.make_async_copy(y_hbm.at[pl.ds(0, CK)], gbuf.at[0], sem).wait()

        def drain_ring(sem):
            pltpu.make_async_copy(src_hbm.at[pl.ds(0, 1)], src_v.at[0], sem).wait()

        @pl.when(s < NS - 1)
        def _():
            pltpu.sync_copy(y_hbm.at[pl.ds(c * HN + s * yrows, yrows)],
                            y_sh.at[pl.ds(s * yrows, yrows)])

        @pl.when(s == NS - 1)
        def _():
            pltpu.sync_copy(y_hbm.at[pl.ds(c * HN + (NS - 1) * yrows, ylast)],
                            y_sh.at[pl.ds((NS - 1) * yrows, ylast)])
            pltpu.sync_copy(zero_hbm.at[pl.ds(0, 8)], y_sh.at[pl.ds(HN, 8)])

        @pl.when(c == 0)
        def _():
            @pl.when(s < NS - 1)
            def _():
                pltpu.sync_copy(z_hbm.at[pl.ds(s * rows, rows)],
                                acc_sh.at[pl.ds(s * rows, rows)])

            @pl.when(s == NS - 1)
            def _():
                pltpu.sync_copy(z_hbm.at[pl.ds((NS - 1) * rows, last)],
                                acc_sh.at[pl.ds((NS - 1) * rows, last)])

        @pl.when((c == 1) & (s == 0))
        def _():
            pltpu.sync_copy(zero_hbm, acc_sh.at[pl.ds(0, 128)])

        plsc.subcore_barrier()

        @pl.when(c == 1)
        def _():
            nfull, rem = divmod(rows, 128)
            for t in range(nfull):
                @pl.when((s > 0) | (t > 0))
                def _():
                    pltpu.sync_copy(acc_sh.at[pl.ds(0, 128)],
                                    acc_sh.at[pl.ds(s * rows + t * 128, 128)])
            if rem:
                @pl.when(s < NS - 1)
                def _():
                    pltpu.sync_copy(acc_sh.at[pl.ds(0, rem)],
                                    acc_sh.at[pl.ds(s * rows + nfull * 128, rem)])
            lrem = last - (last // 128) * 128
            if lrem:
                @pl.when(s == NS - 1)
                def _():
                    pltpu.sync_copy(acc_sh.at[pl.ds(0, lrem)],
                                    acc_sh.at[pl.ds((NS - 1) * rows + (last // 128) * 128, lrem)])

        pltpu.async_copy(src_hbm.at[pl.ds(base, 1)], src_v.at[0], rsem)
        pltpu.async_copy(dst_hbm.at[pl.ds(base, 1)], dst_v.at[0], rsem)
        plsc.subcore_barrier()

        lo = c * HN

        def remap(p):
            for r in range(GC):
                for k in range(CK // 16):
                    v = dst_v[p, 0, r, pl.ds(k * 16, 16)]
                    inr = (v >= lo) & (v < lo + HN)
                    dst_v[p, 0, r, pl.ds(k * 16, 16)] = jnp.where(inr, v - lo, HN)

        drain_ring(rsem)
        drain_ring(rsem)
        remap(0)
        pltpu.async_copy(y_sh.at[dst_v.at[0, 0, 0]], gbuf.at[0], gsem)

        def group(g, carry):
            p = lax.rem(g, 2)
            pn = lax.rem(g + 1, 2)

            @pl.when(g + 1 < GROUPS)
            def _():
                pltpu.async_copy(src_hbm.at[pl.ds(base + g + 1, 1)], src_v.at[pn], rsem)
                pltpu.async_copy(dst_hbm.at[pl.ds(base + g + 1, 1)], dst_v.at[pn], rsem)

            for j in range(GC):
                if j == 0:
                    @pl.when(g > 0)
                    def _():
                        drain_buf(ssem)
                else:
                    drain_buf(ssem)
                if j + 1 < GC:
                    pltpu.async_copy(y_sh.at[dst_v.at[p, 0, j + 1]],
                                     gbuf.at[(j + 1) % 2], gsem)
                drain_buf(gsem)
                pltpu.async_copy(gbuf.at[j % 2], acc_sh.at[src_v.at[p, 0, j]],
                                 ssem, add=True)

            @pl.when(g + 1 < GROUPS)
            def _():
                drain_ring(rsem)
                drain_ring(rsem)
                remap(pn)
                pltpu.async_copy(y_sh.at[dst_v.at[pn, 0, 0]], gbuf.at[0], gsem)

            return carry

        lax.fori_loop(0, GROUPS, group, 0)
        drain_buf(ssem)
        plsc.subcore_barrier()

        @pl.when(s < NS - 1)
        def _():
            pltpu.sync_copy(acc_sh.at[pl.ds(s * rows, rows)],
                            out_hbm.at[c, pl.ds(s * rows, rows)])

        @pl.when(s == NS - 1)
        def _():
            pltpu.sync_copy(acc_sh.at[pl.ds((NS - 1) * rows, last)],
                            out_hbm.at[c, pl.ds((NS - 1) * rows, last)])

    return sc_fn


def kernel(x, edge_index, weight, self_loops, bias):
    N, D = x.shape
    E = edge_index.shape[0]
    GROUPS = -(-E // (NS * GC * CK))
    EP = NS * GROUPS * GC * CK
    pad = EP - E

    y, z = _tc_prep(x, weight, self_loops, bias)

    src = edge_index[:, 0]
    dst = edge_index[:, 1]
    srcp = jnp.concatenate([src, jnp.zeros((pad,), jnp.int32)]).reshape(NS * GROUPS, GC, CK)
    dstp = jnp.concatenate([dst, jnp.full((pad,), N, jnp.int32)]).reshape(NS * GROUPS, GC, CK)
    zero_blk = jnp.zeros((128, D), jnp.float32)

    out_sc = _make_sc_kernel(N, D, GROUPS)(y, z, zero_blk, srcp, dstp)
    return out_sc[0] + out_sc[1]
